# Initial kernel scaffold; baseline (speedup 1.0000x reference)
import jax
import jax.numpy as jnp
from jax.experimental import pallas as pl


def kernel(query, keys, values, reverse):
    raise NotImplementedError("write your pallas kernel here")


# speedup vs baseline: 63.4688x; 63.4688x over previous
import functools

import jax
import jax.numpy as jnp
from jax import lax
from jax.experimental import pallas as pl
from jax.experimental.pallas import tpu as pltpu
from jax.experimental.pallas import tpu_sc as plsc

K_TOP = 32
GSZ = 128
MBLK = 2048
NEG = -3.0e38
QTILE = 256


def _p1_body(m_real, q_ref, k_ref, s_ref, g_ref):
    i = pl.program_id(0)
    q = q_ref[...]
    k = k_ref[...]
    s = lax.dot_general(q, k, (((1,), (1,)), ((), ())),
                        preferred_element_type=jnp.float32)
    s = s * (q.shape[1] ** -0.5)
    col = lax.broadcasted_iota(jnp.int32, s.shape, 1) + i * MBLK
    s = jnp.where(col < m_real, s, NEG)
    s_ref[...] = s
    s3 = s.reshape(s.shape[0], MBLK // GSZ, GSZ)
    g_ref[0] = jnp.max(s3, axis=2)


def _p1(q, kpad, m_real):
    Q, D = q.shape
    Mp = kpad.shape[0]
    nblk = Mp // MBLK
    return pl.pallas_call(
        functools.partial(_p1_body, m_real),
        grid=(nblk,),
        in_specs=[
            pl.BlockSpec((Q, D), lambda i: (0, 0)),
            pl.BlockSpec((MBLK, D), lambda i: (i, 0)),
        ],
        out_specs=[
            pl.BlockSpec((Q, MBLK), lambda i: (0, i)),
            pl.BlockSpec((1, Q, MBLK // GSZ), lambda i: (i, 0, 0)),
        ],
        out_shape=[
            jax.ShapeDtypeStruct((Q, Mp), jnp.float32),
            jax.ShapeDtypeStruct((nblk, Q, MBLK // GSZ), jnp.float32),
        ],
    )(q, kpad)


def _p2_body(g_ref, rowid_ref, gid_ref):
    g = g_ref[...]
    Q, G = g.shape
    col = lax.broadcasted_iota(jnp.int32, (Q, G), 1)
    rowq = lax.broadcasted_iota(jnp.int32, (Q, 1), 0)
    big = jnp.int32(2 ** 30)
    for j in range(K_TOP):
        m = jnp.max(g, axis=1, keepdims=True)
        pos = jnp.min(jnp.where(g == m, col, big), axis=1, keepdims=True)
        gid_ref[:, j:j + 1] = pos
        rowid_ref[:, j:j + 1] = rowq * G + pos
        g = jnp.where(col == pos, NEG, g)


def _p2(gmax):
    Q, G = gmax.shape
    return pl.pallas_call(
        _p2_body,
        out_shape=[
            jax.ShapeDtypeStruct((Q, K_TOP), jnp.int32),
            jax.ShapeDtypeStruct((Q, K_TOP), jnp.int32),
        ],
    )(gmax)


def _sc_gather(table, idx):
    N = idx.shape[0]
    D = table.shape[1]
    info = plsc.get_sparse_core_info()
    nw = info.num_cores * info.num_subcores
    bpw = N // nw
    ch = 128
    nch = bpw // ch
    mesh = plsc.VectorSubcoreMesh(core_axis_name="c", subcore_axis_name="s")

    @functools.partial(
        pl.kernel, mesh=mesh,
        out_type=jax.ShapeDtypeStruct((N, D), jnp.float32),
        scratch_types=[
            pltpu.VMEM((ch,), jnp.int32),
            pltpu.VMEM((ch, D), jnp.float32),
            pltpu.SemaphoreType.DMA,
        ],
    )
    def k(table_hbm, idx_hbm, out_hbm, idx_v, rows_v, sem):
        wid = lax.axis_index("s") * info.num_cores + lax.axis_index("c")
        for c in range(nch):
            base = wid * bpw + c * ch
            pltpu.sync_copy(idx_hbm.at[pl.ds(base, ch)], idx_v)
            pltpu.async_copy(table_hbm.at[idx_v], rows_v, sem).wait()
            pltpu.sync_copy(rows_v, out_hbm.at[pl.ds(base, ch)])

    return k(table, idx)


def _p4_body(c_ref, gid_ref, w_ref, kidx_ref, kfull_ref):
    Qt = c_ref.shape[0]
    W = c_ref.shape[1]
    lane = lax.broadcasted_iota(jnp.int32, (Qt, GSZ), 1)
    for j in range(K_TOP):
        kfull_ref[:, j * GSZ:(j + 1) * GSZ] = gid_ref[:, j:j + 1] * GSZ + lane
    kidx_all = kfull_ref[...]
    s = c_ref[...]
    col = lax.broadcasted_iota(jnp.int32, (Qt, W), 1)
    big = jnp.int32(2 ** 30)
    for j in range(K_TOP):
        m = jnp.max(s, axis=1, keepdims=True)
        sel = s == m
        pos = jnp.min(jnp.where(sel, col, big), axis=1, keepdims=True)
        hit = col == pos
        iv = jnp.max(jnp.where(hit, kidx_all, 0), axis=1, keepdims=True)
        w_ref[:, j:j + 1] = m
        kidx_ref[:, j:j + 1] = iv
        s = jnp.where(hit, NEG, s)
    sr = w_ref[...]
    e = jnp.exp(sr - sr[:, 0:1])
    w_ref[...] = e / jnp.sum(e, axis=1, keepdims=True)


def _p4(cand, gids):
    Q, W = cand.shape
    nt = Q // QTILE
    return pl.pallas_call(
        _p4_body,
        grid=(nt,),
        in_specs=[
            pl.BlockSpec((QTILE, W), lambda i: (i, 0)),
            pl.BlockSpec((QTILE, K_TOP), lambda i: (i, 0)),
        ],
        out_specs=[
            pl.BlockSpec((QTILE, K_TOP), lambda i: (i, 0)),
            pl.BlockSpec((QTILE, K_TOP), lambda i: (i, 0)),
        ],
        out_shape=[
            jax.ShapeDtypeStruct((Q, K_TOP), jnp.float32),
            jax.ShapeDtypeStruct((Q, K_TOP), jnp.int32),
        ],
        scratch_shapes=[pltpu.VMEM((QTILE, W), jnp.int32)],
    )(cand, gids)


def _p5_body(g_ref, w_ref, o_ref):
    acc = w_ref[:, 0:1] * g_ref[:, 0:GSZ]
    for j in range(1, K_TOP):
        acc = acc + w_ref[:, j:j + 1] * g_ref[:, j * GSZ:(j + 1) * GSZ]
    o_ref[...] = acc


def _p5(gv, w):
    Q = w.shape[0]
    nt = Q // QTILE
    return pl.pallas_call(
        _p5_body,
        grid=(nt,),
        in_specs=[
            pl.BlockSpec((QTILE, K_TOP * GSZ), lambda i: (i, 0)),
            pl.BlockSpec((QTILE, K_TOP), lambda i: (i, 0)),
        ],
        out_specs=pl.BlockSpec((QTILE, GSZ), lambda i: (i, 0)),
        out_shape=jax.ShapeDtypeStruct((Q, GSZ), jnp.float32),
    )(gv, w)


def kernel(query, keys, values, reverse):
    B, C, D = query.shape
    M = keys.shape[0]
    Q = B * C
    sign = jnp.where(reverse, -1.0, 1.0).astype(jnp.float32)
    q = query.reshape(Q, D) * sign

    Mp = ((M + MBLK - 1) // MBLK) * MBLK
    kpad = jnp.pad(keys, ((0, Mp - M), (0, 0)))
    G = Mp // GSZ

    S, gmax3 = _p1(q, kpad, M)
    gmax = gmax3.transpose(1, 0, 2).reshape(Q, G)
    rowids, gids = _p2(gmax)
    cand = _sc_gather(S.reshape(Q * G, GSZ), rowids.reshape(-1))
    w, kidx = _p4(cand.reshape(Q, K_TOP * GSZ), gids)
    gv = _sc_gather(values, kidx.reshape(-1))
    out = _p5(gv.reshape(Q, K_TOP * GSZ), w)
    return (out * sign).reshape(B, C, D)
